# Initial kernel scaffold; baseline (speedup 1.0000x reference)
import jax
import jax.numpy as jnp
from jax.experimental import pallas as pl


def kernel(ingredients, mask, table):
    raise NotImplementedError("write your pallas kernel here")


# speedup vs baseline: 2.8154x; 2.8154x over previous
import functools
import jax
import jax.numpy as jnp
from jax import lax
from jax.experimental import pallas as pl
from jax.experimental.pallas import tpu as pltpu
from jax.experimental.pallas import tpu_sc as plsc

B = 16384
H = 50
HP = 64
D = 32
L = 16
NW = 32
BPW = B // NW
RPC = 2
IPC = RPC * H
NCHUNK = BPW // RPC
NBUF = 4


def _sc_body(table_hbm, idx_hbm, mask_hbm, out_hbm,
             idx_v, mask_v, rows_v, out_v, sem0, sem1, sem2, sem3):
    sems = [sem0, sem1, sem2, sem3]
    wid = lax.axis_index("s") * 2 + lax.axis_index("c")
    row0 = wid * NCHUNK

    pltpu.sync_copy(idx_hbm.at[pl.ds(row0, NCHUNK)], idx_v)
    pltpu.sync_copy(mask_hbm.at[pl.ds(row0, NCHUNK)], mask_v)

    def issue(j, b):
        pltpu.make_async_copy(table_hbm.at[idx_v.at[j]], rows_v.at[b],
                              sems[b]).start()

    def drain(b):
        pltpu.make_async_copy(table_hbm.at[idx_v.at[0]], rows_v.at[b],
                              sems[b]).wait()

    for b in range(NBUF):
        issue(b, b)

    def chunk_compute(j, b):
        for r in range(RPC):
            mbase = r * HP
            m0 = mask_v[j, pl.ds(mbase, L)]
            m1 = mask_v[j, pl.ds(mbase + L, L)]
            m2 = mask_v[j, pl.ds(mbase + 2 * L, L)]
            m3 = mask_v[j, pl.ds(mbase + 3 * L, L)]
            mv = (m0, m1, m2, m3)
            d = jnp.float32(0.0)
            acc0 = jnp.zeros((L,), jnp.float32)
            acc1 = jnp.zeros((L,), jnp.float32)
            for l in range(H):
                m = mv[l // L][l % L]
                d = d + m
                acc0 = acc0 + m * rows_v[b, r * H + l, pl.ds(0, L)]
                acc1 = acc1 + m * rows_v[b, r * H + l, pl.ds(L, L)]
            dv = jnp.maximum(jnp.broadcast_to(d, (L,)), 1.0)
            inv = jnp.ones((L,), jnp.float32) / dv
            orow = RPC * j + r
            out_v[orow, pl.ds(0, L)] = acc0 * inv
            out_v[orow, pl.ds(L, L)] = acc1 * inv

    def g_body(g, carry):
        for b in range(NBUF):
            j = g * NBUF + b
            drain(b)
            chunk_compute(j, b)
            nj = j + NBUF

            @pl.when(nj < NCHUNK)
            def _():
                issue(nj, b)
        return carry

    lax.fori_loop(0, NCHUNK // NBUF, g_body, 0)
    pltpu.sync_copy(out_v, out_hbm.at[pl.ds(wid * BPW, BPW)])


@functools.partial(jax.jit, donate_argnums=())
def _run(table, idx2, mask2):
    mesh = plsc.VectorSubcoreMesh(core_axis_name="c", subcore_axis_name="s")
    f = pl.kernel(
        _sc_body,
        out_type=jax.ShapeDtypeStruct((B, D), jnp.float32),
        mesh=mesh,
        compiler_params=pltpu.CompilerParams(use_tc_tiling_on_sc=False),
        scratch_types=[
            pltpu.VMEM((NCHUNK, IPC), jnp.int32),
            pltpu.VMEM((NCHUNK, RPC * HP), jnp.float32),
            pltpu.VMEM((NBUF, IPC, D), jnp.float32),
            pltpu.VMEM((BPW, D), jnp.float32),
            pltpu.SemaphoreType.DMA,
            pltpu.SemaphoreType.DMA,
            pltpu.SemaphoreType.DMA,
            pltpu.SemaphoreType.DMA,
        ],
    )
    return f(table, idx2, mask2)


def kernel(ingredients, mask, table):
    idx2 = ingredients.reshape(B * H // IPC, IPC)
    mask_p = jnp.pad(mask, ((0, 0), (0, HP - H)))
    mask2 = mask_p.reshape(B * HP // (RPC * HP), RPC * HP)
    return _run(table, idx2, mask2)
